# baseline (speedup 1.0000x reference)
import jax
import jax.numpy as jnp
import numpy as np
from jax.experimental import pallas as pl
from jax.experimental.pallas import tpu as pltpu

ROWS = 128
COLS = 100000
BLOCK_COLS = 2048

_TINY = np.float32(np.finfo(np.float32).tiny)
_KS1 = np.uint32(42)
_KS2 = np.uint32(42 ^ 0x1BD11BDA)


def _rotl(x, d):
    return (x << np.uint32(d)) | (x >> np.uint32(32 - d))


def _round4(x0, x1, rots):
    for r in rots:
        x0 = x0 + x1
        x1 = _rotl(x1, r)
        x1 = x1 ^ x0
    return x0, x1


def _threefry_bits(m):
    x0 = m
    x1 = _rotl(m, 13) ^ m
    x0, x1 = _round4(x0, x1, (15, 26, 6))
    x0, x1 = x0 + _KS1, x1 + (_KS2 + np.uint32(1))
    x0, x1 = _round4(x0, x1, (17, 29, 16, 24))
    x0, x1 = x0 + _KS2, x1 + np.uint32(2)
    x0, x1 = _round4(x0, x1, (13, 15, 26, 6))
    x1 = x1 + (_KS1 + np.uint32(3))
    x0, x1 = _round4(x0, x1, (17, 29, 16, 24))
    x0, x1 = x0 + _KS1, x1 + (_KS2 + np.uint32(4))
    x0, x1 = _round4(x0, x1, (13, 15, 26, 6))
    x0, x1 = x0 + _KS2, x1 + np.uint32(5)
    return x0 ^ x1


def _sample_kernel(logits_ref, out_ref, acc_val, acc_idx):
    j = pl.program_id(0)
    nblocks = pl.num_programs(0)

    lane = jax.lax.broadcasted_iota(jnp.int32, (ROWS, BLOCK_COLS), 1)
    row = jax.lax.broadcasted_iota(jnp.int32, (ROWS, BLOCK_COLS), 0)
    col = j * BLOCK_COLS + lane
    m = (row * COLS + col + 42).astype(jnp.uint32)

    bits = _threefry_bits(m)

    fbits = (bits >> np.uint32(9)) | np.uint32(0x3F800000)
    floats = jax.lax.bitcast_convert_type(fbits, jnp.float32) - np.float32(1.0)
    u = floats + _TINY
    g = -jnp.log(-jnp.log(u))

    x = logits_ref[...] + g
    x = jnp.where(col < COLS, x, -jnp.inf)

    @pl.when(j == 0)
    def _():
        acc_val[...] = x
        acc_idx[...] = col

    @pl.when(j > 0)
    def _():
        av = acc_val[...]
        upd = x > av
        acc_val[...] = jnp.where(upd, x, av)
        acc_idx[...] = jnp.where(upd, col, acc_idx[...])

    @pl.when(j == nblocks - 1)
    def _():
        av = acc_val[...]
        bm = jnp.max(av, axis=1, keepdims=True)
        out_ref[...] = jnp.min(
            jnp.where(av == bm, acc_idx[...], jnp.int32(2**30)),
            axis=1, keepdims=True)


@jax.jit
def kernel(logits):
    nblocks = pl.cdiv(COLS, BLOCK_COLS)
    out = pl.pallas_call(
        _sample_kernel,
        grid=(nblocks,),
        in_specs=[pl.BlockSpec((ROWS, BLOCK_COLS), lambda j: (0, j))],
        out_specs=pl.BlockSpec((ROWS, 1), lambda j: (0, 0)),
        out_shape=jax.ShapeDtypeStruct((ROWS, 1), jnp.int32),
        scratch_shapes=[
            pltpu.VMEM((ROWS, BLOCK_COLS), jnp.float32),
            pltpu.VMEM((ROWS, BLOCK_COLS), jnp.int32),
        ],
    )(logits)
    return out.reshape(ROWS)


# speedup vs baseline: 1.5176x; 1.5176x over previous
import jax
import jax.numpy as jnp
import numpy as np
from jax.experimental import pallas as pl
from jax.experimental.pallas import tpu as pltpu

ROWS = 128
COLS = 100000
BLOCK_COLS = 2048

_TINY = np.float32(np.finfo(np.float32).tiny)
_KS1 = np.uint32(42)
_KS2 = np.uint32(42 ^ 0x1BD11BDA)


def _rotl(x, d):
    return (x << np.uint32(d)) | (x >> np.uint32(32 - d))


def _round4(x0, x1, rots):
    for r in rots:
        x0 = x0 + x1
        x1 = _rotl(x1, r)
        x1 = x1 ^ x0
    return x0, x1


def _threefry_bits(m):
    x0 = m
    x1 = _rotl(m, 13) ^ m
    x0, x1 = _round4(x0, x1, (15, 26, 6))
    x0, x1 = x0 + _KS1, x1 + (_KS2 + np.uint32(1))
    x0, x1 = _round4(x0, x1, (17, 29, 16, 24))
    x0, x1 = x0 + _KS2, x1 + np.uint32(2)
    x0, x1 = _round4(x0, x1, (13, 15, 26, 6))
    x1 = x1 + (_KS1 + np.uint32(3))
    x0, x1 = _round4(x0, x1, (17, 29, 16, 24))
    x0, x1 = x0 + _KS1, x1 + (_KS2 + np.uint32(4))
    x0, x1 = _round4(x0, x1, (13, 15, 26, 6))
    x0, x1 = x0 + _KS2, x1 + np.uint32(5)
    return x0 ^ x1


def _sample_kernel(logits_ref, out_ref, best_val, best_idx):
    j = pl.program_id(0)
    nblocks = pl.num_programs(0)

    lane = jax.lax.broadcasted_iota(jnp.int32, (ROWS, BLOCK_COLS), 1)
    row = jax.lax.broadcasted_iota(jnp.int32, (ROWS, BLOCK_COLS), 0)
    col = j * BLOCK_COLS + lane
    m = (row * COLS + col + 42).astype(jnp.uint32)

    bits = _threefry_bits(m)

    fbits = (bits >> np.uint32(9)) | np.uint32(0x3F800000)
    floats = jax.lax.bitcast_convert_type(fbits, jnp.float32) - np.float32(1.0)
    u = floats + _TINY
    g = -jnp.log(-jnp.log(u))

    x = logits_ref[...] + g
    x = jnp.where(col < COLS, x, -jnp.inf)

    bm = jnp.max(x, axis=1, keepdims=True)
    bi = jnp.min(jnp.where(x == bm, col, jnp.int32(2**30)), axis=1,
                 keepdims=True)

    @pl.when(j == 0)
    def _():
        best_val[...] = bm
        best_idx[...] = bi

    @pl.when(j > 0)
    def _():
        upd = bm > best_val[...]
        best_val[...] = jnp.where(upd, bm, best_val[...])
        best_idx[...] = jnp.where(upd, bi, best_idx[...])

    @pl.when(j == nblocks - 1)
    def _():
        out_ref[...] = best_idx[...]


@jax.jit
def kernel(logits):
    nblocks = pl.cdiv(COLS, BLOCK_COLS)
    out = pl.pallas_call(
        _sample_kernel,
        grid=(nblocks,),
        in_specs=[pl.BlockSpec((ROWS, BLOCK_COLS), lambda j: (0, j))],
        out_specs=pl.BlockSpec((ROWS, 1), lambda j: (0, 0)),
        out_shape=jax.ShapeDtypeStruct((ROWS, 1), jnp.int32),
        scratch_shapes=[
            pltpu.VMEM((ROWS, 1), jnp.float32),
            pltpu.VMEM((ROWS, 1), jnp.int32),
        ],
    )(logits)
    return out.reshape(ROWS)
